# baseline (speedup 1.0000x reference)
import jax
import jax.numpy as jnp
from jax.experimental import pallas as pl
from jax.experimental.pallas import tpu as pltpu

_BM = 200
_BK = 2500


def _gcn_kernel(x_ref, a_ref, w_ref, o_ref):
    n = a_ref.shape[1]
    acc = jnp.zeros(o_ref.shape, dtype=jnp.float32)
    for k0 in range(0, n, _BK):
        a16 = a_ref[:, k0:k0 + _BK].astype(jnp.bfloat16)
        acc += jax.lax.dot_general(
            a16, x_ref[k0:k0 + _BK, :],
            dimension_numbers=(((1,), (0,)), ((), ())),
            preferred_element_type=jnp.float32,
        )
    o_ref[...] = acc * w_ref[...]


def kernel(input, A, W):
    n, d = A.shape[0], input.shape[1]
    w2 = W.reshape(1, d)
    x16 = input.astype(jnp.bfloat16)
    return pl.pallas_call(
        _gcn_kernel,
        grid=(n // _BM,),
        in_specs=[
            pl.BlockSpec((n, d), lambda i: (0, 0)),
            pl.BlockSpec((_BM, n), lambda i: (i, 0)),
            pl.BlockSpec((1, d), lambda i: (0, 0)),
        ],
        out_specs=pl.BlockSpec((_BM, d), lambda i: (i, 0)),
        out_shape=jax.ShapeDtypeStruct((n, d), jnp.float32),
        compiler_params=pltpu.CompilerParams(
            dimension_semantics=("parallel",),
        ),
    )(x16, A, w2)
